# baseline (speedup 1.0000x reference)
import functools

import jax
import jax.numpy as jnp
from jax import lax
from jax.experimental import pallas as pl
from jax.experimental.pallas import tpu as pltpu
from jax.experimental.pallas import tpu_sc as plsc

USERS = 50000
HALF = 32
NC = 2
NS = 16
GRP = 112


NBUF = 6
KLEAD = 3


def _spmm_halves(idx_g, vals_g, tab_lo, tab_hi, init_lo, init_hi):
    groups = idx_g.shape[0]
    assert groups % (NS * NBUF) == 0
    tgroups = groups // NS
    stripe = 3128
    last_r0 = USERS - stripe

    mesh = plsc.VectorSubcoreMesh(
        core_axis_name="c", subcore_axis_name="s", num_cores=NC,
        num_subcores=NS)

    def body(idx_hbm, vals_hbm, tlo_hbm, thi_hbm, ilo_hbm, ihi_hbm,
             olo_hbm, ohi_hbm, acc, ebufs, vbufs, dstbs, rowss,
             esem, gsem, ssem):
        c = lax.axis_index("c")
        s = lax.axis_index("s")

        def gather_issue(src_idx_ref, rows_ref):
            @pl.when(c == 0)
            def _():
                pltpu.async_copy(tlo_hbm.at[src_idx_ref], rows_ref, gsem)

            @pl.when(c != 0)
            def _():
                pltpu.async_copy(thi_hbm.at[src_idx_ref], rows_ref, gsem)

        r0 = pl.multiple_of(jnp.minimum(s * stripe, last_r0), 8)

        @pl.when(c == 0)
        def _():
            pltpu.sync_copy(ilo_hbm.at[pl.ds(r0, stripe)],
                            acc.at[pl.ds(r0, stripe)])

        @pl.when(c != 0)
        def _():
            pltpu.sync_copy(ihi_hbm.at[pl.ds(r0, stripe)],
                            acc.at[pl.ds(r0, stripe)])

        plsc.subcore_barrier()

        g0 = s * tgroups

        for i in range(NBUF):
            pltpu.async_copy(idx_hbm.at[g0 + i], ebufs[i], esem)
            pltpu.async_copy(vals_hbm.at[g0 + i], vbufs[i], esem)
        for i in range(KLEAD):
            pltpu.make_async_copy(idx_hbm.at[g0 + i], ebufs[i], esem).wait()
            pltpu.make_async_copy(vals_hbm.at[g0 + i], vbufs[i], esem).wait()
            gather_issue(ebufs[i].at[1], rowss[i])

        def outer_body(tb, carry):
            for b in range(NBUF):
                t = tb * NBUF + b
                g = g0 + t
                bn = (b + KLEAD) % NBUF

                @pl.when(t + KLEAD < tgroups)
                def _():
                    @pl.when(t + KLEAD >= NBUF)
                    def _():
                        pltpu.make_async_copy(
                            rowss[bn], acc.at[dstbs[bn].at[0]], ssem).wait()
                    pltpu.make_async_copy(
                        idx_hbm.at[g + KLEAD], ebufs[bn], esem).wait()
                    pltpu.make_async_copy(
                        vals_hbm.at[g + KLEAD], vbufs[bn], esem).wait()
                    gather_issue(ebufs[bn].at[1], rowss[bn])

                pltpu.make_async_copy(
                    tlo_hbm.at[ebufs[b].at[1]], rowss[b], gsem).wait()

                def sub_body(k, carry2):
                    sl = pl.ds(k * 16, 16)
                    v16 = vbufs[b][0, sl]
                    dstbs[b][0, sl] = ebufs[b][0, sl]
                    for j in range(16):
                        spl = v16.at[jnp.full((16,), j, jnp.int32)].get(
                            mode="promise_in_bounds")
                        e = k * 16 + j
                        rowss[b][e, pl.ds(0, 16)] = (
                            rowss[b][e, pl.ds(0, 16)] * spl)
                        rowss[b][e, pl.ds(16, 16)] = (
                            rowss[b][e, pl.ds(16, 16)] * spl)
                    return carry2

                lax.fori_loop(0, GRP // 16, sub_body, 0, unroll=False)

                pltpu.async_copy(
                    rowss[b], acc.at[dstbs[b].at[0]], ssem, add=True)

                @pl.when(t + NBUF < tgroups)
                def _():
                    pltpu.async_copy(idx_hbm.at[g + NBUF], ebufs[b], esem)
                    pltpu.async_copy(vals_hbm.at[g + NBUF], vbufs[b], esem)
            return carry

        lax.fori_loop(0, tgroups // NBUF, outer_body, 0, unroll=False)

        for b in range(NBUF):
            pltpu.make_async_copy(
                rowss[b], acc.at[dstbs[b].at[0]], ssem).wait()

        plsc.subcore_barrier()

        @pl.when(c == 0)
        def _():
            pltpu.sync_copy(acc.at[pl.ds(r0, stripe)],
                            olo_hbm.at[pl.ds(r0, stripe)])

        @pl.when(c != 0)
        def _():
            pltpu.sync_copy(acc.at[pl.ds(r0, stripe)],
                            ohi_hbm.at[pl.ds(r0, stripe)])

    out_type = [jax.ShapeDtypeStruct((USERS, HALF), jnp.float32)] * 2
    f = pl.kernel(
        body,
        out_type=out_type,
        mesh=mesh,
        scratch_types=[
            pltpu.VMEM_SHARED((USERS, HALF), jnp.float32),
            [pltpu.VMEM((2, GRP), jnp.int32)] * NBUF,
            [pltpu.VMEM((1, GRP), jnp.float32)] * NBUF,
            [pltpu.VMEM((1, GRP), jnp.int32)] * NBUF,
            [pltpu.VMEM((GRP, HALF), jnp.float32)] * NBUF,
            pltpu.SemaphoreType.DMA,
            pltpu.SemaphoreType.DMA,
            pltpu.SemaphoreType.DMA,
        ],
        compiler_params=pltpu.CompilerParams(use_tc_tiling_on_sc=False),
    )
    return f(idx_g, vals_g, tab_lo, tab_hi, init_lo, init_hi)


def _mm_body(alo, ahi, ulo, uhi, wa, wb, wc, wd, olo, ohi):
    acc = jnp.dot(alo[...], wa[...], preferred_element_type=jnp.float32)
    acc = acc + jnp.dot(ahi[...], wb[...], preferred_element_type=jnp.float32)
    acc = acc + jnp.dot(ulo[...], wc[...], preferred_element_type=jnp.float32)
    acc = acc + jnp.dot(uhi[...], wd[...], preferred_element_type=jnp.float32)
    olo[...] = acc[:, :HALF]
    ohi[...] = acc[:, HALF:]


def _layer_matmul(a_lo, a_hi, u_lo, u_hi, w):
    rows_blk = 2000
    grid = (USERS // rows_blk,)
    in_spec_x = pl.BlockSpec((rows_blk, HALF), lambda i: (i, 0))
    in_spec_w = pl.BlockSpec((HALF, 2 * HALF), lambda i: (0, 0))
    out_spec = pl.BlockSpec((rows_blk, HALF), lambda i: (i, 0))
    wa, wb, wc, wd = (w[0:32], w[32:64], w[64:96], w[96:128])
    return pl.pallas_call(
        _mm_body,
        grid=grid,
        in_specs=[in_spec_x] * 4 + [in_spec_w] * 4,
        out_specs=[out_spec] * 2,
        out_shape=[jax.ShapeDtypeStruct((USERS, HALF), jnp.float32)] * 2,
    )(a_lo, a_hi, u_lo, u_hi, wa, wb, wc, wd)


def _pack_edges(idx, vals):
    e = vals.shape[0]
    per = GRP * NS * NBUF
    e_pad = ((e + per - 1) // per) * per
    pad = e_pad - e
    dst = jnp.concatenate([idx[0], jnp.zeros((pad,), jnp.int32)])
    src = jnp.concatenate([idx[1], jnp.zeros((pad,), jnp.int32)])
    val = jnp.concatenate([vals, jnp.zeros((pad,), jnp.float32)])
    idx_g = jnp.stack([dst.reshape(-1, GRP), src.reshape(-1, GRP)], axis=1)
    return idx_g, val.reshape(-1, 1, GRP)


def kernel(user_emb, item_emb, weight0, weight1, social_vals, inter_vals,
           social_idx, inter_idx):
    ue_lo, ue_hi = user_emb[:, :HALF], user_emb[:, HALF:]
    it_lo, it_hi = item_emb[:, :HALF], item_emb[:, HALF:]
    zeros = jnp.zeros((USERS, HALF), jnp.float32)

    soc_i, soc_v = _pack_edges(social_idx, social_vals)
    int_i, int_v = _pack_edges(inter_idx, inter_vals)

    a_lo, a_hi = _spmm_halves(soc_i, soc_v, ue_lo, ue_hi, zeros, zeros)
    u_lo, u_hi = _layer_matmul(a_lo, a_hi, ue_lo, ue_hi, weight0)
    a_lo, a_hi = _spmm_halves(soc_i, soc_v, u_lo, u_hi, zeros, zeros)
    u_lo, u_hi = _layer_matmul(a_lo, a_hi, u_lo, u_hi, weight1)
    f_lo, f_hi = _spmm_halves(int_i, int_v, it_lo, it_hi, u_lo, u_hi)

    final_user = jnp.concatenate([f_lo, f_hi], axis=1)
    return (final_user, item_emb)


# speedup vs baseline: 1.0350x; 1.0350x over previous
import functools

import jax
import jax.numpy as jnp
from jax import lax
from jax.experimental import pallas as pl
from jax.experimental.pallas import tpu as pltpu
from jax.experimental.pallas import tpu_sc as plsc

USERS = 50000
HALF = 32
NC = 2
NS = 16
GRP = 128


NBUF = 5
KLEAD = 3


def _spmm_halves(dst_e, src_e, vals_e, tab_lo, tab_hi, init_lo, init_hi):
    groups = dst_e.shape[0] // GRP
    assert dst_e.shape[0] % (GRP * NS * NBUF) == 0
    tgroups = groups // NS
    stripe = 3128
    last_r0 = USERS - stripe

    mesh = plsc.VectorSubcoreMesh(
        core_axis_name="c", subcore_axis_name="s", num_cores=NC,
        num_subcores=NS)

    def body(dst_hbm, src_hbm, vals_hbm, tlo_hbm, thi_hbm, ilo_hbm, ihi_hbm,
             olo_hbm, ohi_hbm, acc, ebufs, vbufs, dstbs, rowss,
             esem, gsem, ssem):
        c = lax.axis_index("c")
        s = lax.axis_index("s")

        def gather_issue(src_idx_ref, rows_ref):
            @pl.when(c == 0)
            def _():
                pltpu.async_copy(tlo_hbm.at[src_idx_ref], rows_ref, gsem)

            @pl.when(c != 0)
            def _():
                pltpu.async_copy(thi_hbm.at[src_idx_ref], rows_ref, gsem)

        r0 = pl.multiple_of(jnp.minimum(s * stripe, last_r0), 8)

        @pl.when(c == 0)
        def _():
            pltpu.sync_copy(ilo_hbm.at[pl.ds(r0, stripe)],
                            acc.at[pl.ds(r0, stripe)])

        @pl.when(c != 0)
        def _():
            pltpu.sync_copy(ihi_hbm.at[pl.ds(r0, stripe)],
                            acc.at[pl.ds(r0, stripe)])

        plsc.subcore_barrier()

        g0 = s * tgroups

        def edge_dmas(g, i, issue):
            e0 = pl.multiple_of(g * GRP, GRP)
            dsts = [(dst_hbm.at[pl.ds(e0, GRP)], ebufs[i].at[0]),
                    (src_hbm.at[pl.ds(e0, GRP)], ebufs[i].at[1]),
                    (vals_hbm.at[pl.ds(e0, GRP)], vbufs[i].at[0])]
            for hbm_sl, buf in dsts:
                if issue:
                    pltpu.async_copy(hbm_sl, buf, esem)
                else:
                    pltpu.make_async_copy(hbm_sl, buf, esem).wait()

        for i in range(NBUF):
            edge_dmas(g0 + i, i, True)
        for i in range(KLEAD):
            edge_dmas(g0 + i, i, False)
            gather_issue(ebufs[i].at[1], rowss[i])

        def outer_body(tb, carry):
            for b in range(NBUF):
                t = tb * NBUF + b
                g = g0 + t
                bn = (b + KLEAD) % NBUF

                @pl.when(t + KLEAD < tgroups)
                def _():
                    @pl.when(t + KLEAD >= NBUF)
                    def _():
                        pltpu.make_async_copy(
                            rowss[bn], acc.at[dstbs[bn].at[0]], ssem).wait()
                    edge_dmas(g + KLEAD, bn, False)
                    gather_issue(ebufs[bn].at[1], rowss[bn])

                pltpu.make_async_copy(
                    tlo_hbm.at[ebufs[b].at[1]], rowss[b], gsem).wait()

                def sub_body(k, carry2):
                    sl = pl.ds(k * 16, 16)
                    v16 = vbufs[b][0, sl]
                    dstbs[b][0, sl] = ebufs[b][0, sl]
                    for j in range(16):
                        spl = v16.at[jnp.full((16,), j, jnp.int32)].get(
                            mode="promise_in_bounds")
                        e = k * 16 + j
                        rowss[b][e, pl.ds(0, 16)] = (
                            rowss[b][e, pl.ds(0, 16)] * spl)
                        rowss[b][e, pl.ds(16, 16)] = (
                            rowss[b][e, pl.ds(16, 16)] * spl)
                    return carry2

                lax.fori_loop(0, GRP // 16, sub_body, 0, unroll=False)

                pltpu.async_copy(
                    rowss[b], acc.at[dstbs[b].at[0]], ssem, add=True)

                @pl.when(t + NBUF < tgroups)
                def _():
                    edge_dmas(g + NBUF, b, True)
            return carry

        lax.fori_loop(0, tgroups // NBUF, outer_body, 0, unroll=False)

        for b in range(NBUF):
            pltpu.make_async_copy(
                rowss[b], acc.at[dstbs[b].at[0]], ssem).wait()

        plsc.subcore_barrier()

        @pl.when(c == 0)
        def _():
            pltpu.sync_copy(acc.at[pl.ds(r0, stripe)],
                            olo_hbm.at[pl.ds(r0, stripe)])

        @pl.when(c != 0)
        def _():
            pltpu.sync_copy(acc.at[pl.ds(r0, stripe)],
                            ohi_hbm.at[pl.ds(r0, stripe)])

    out_type = [jax.ShapeDtypeStruct((USERS, HALF), jnp.float32)] * 2
    f = pl.kernel(
        body,
        out_type=out_type,
        mesh=mesh,
        scratch_types=[
            pltpu.VMEM_SHARED((USERS, HALF), jnp.float32),
            [pltpu.VMEM((2, GRP), jnp.int32)] * NBUF,
            [pltpu.VMEM((1, GRP), jnp.float32)] * NBUF,
            [pltpu.VMEM((1, GRP), jnp.int32)] * NBUF,
            [pltpu.VMEM((GRP, HALF), jnp.float32)] * NBUF,
            pltpu.SemaphoreType.DMA,
            pltpu.SemaphoreType.DMA,
            pltpu.SemaphoreType.DMA,
        ],
        compiler_params=pltpu.CompilerParams(use_tc_tiling_on_sc=False),
    )
    return f(dst_e, src_e, vals_e, tab_lo, tab_hi, init_lo, init_hi)


def _mm_body(alo, ahi, ulo, uhi, wa, wb, wc, wd, olo, ohi):
    acc = jnp.dot(alo[...], wa[...], preferred_element_type=jnp.float32)
    acc = acc + jnp.dot(ahi[...], wb[...], preferred_element_type=jnp.float32)
    acc = acc + jnp.dot(ulo[...], wc[...], preferred_element_type=jnp.float32)
    acc = acc + jnp.dot(uhi[...], wd[...], preferred_element_type=jnp.float32)
    olo[...] = acc[:, :HALF]
    ohi[...] = acc[:, HALF:]


def _layer_matmul(a_lo, a_hi, u_lo, u_hi, w):
    rows_blk = 2000
    grid = (USERS // rows_blk,)
    in_spec_x = pl.BlockSpec((rows_blk, HALF), lambda i: (i, 0))
    in_spec_w = pl.BlockSpec((HALF, 2 * HALF), lambda i: (0, 0))
    out_spec = pl.BlockSpec((rows_blk, HALF), lambda i: (i, 0))
    wa, wb, wc, wd = (w[0:32], w[32:64], w[64:96], w[96:128])
    return pl.pallas_call(
        _mm_body,
        grid=grid,
        in_specs=[in_spec_x] * 4 + [in_spec_w] * 4,
        out_specs=[out_spec] * 2,
        out_shape=[jax.ShapeDtypeStruct((USERS, HALF), jnp.float32)] * 2,
    )(a_lo, a_hi, u_lo, u_hi, wa, wb, wc, wd)


def _pack_edges(idx, vals):
    e = vals.shape[0]
    per = GRP * NS * NBUF
    e_pad = ((e + per - 1) // per) * per
    pad = e_pad - e
    dst = jnp.concatenate([idx[0], jnp.zeros((pad,), jnp.int32)])
    src = jnp.concatenate([idx[1], jnp.zeros((pad,), jnp.int32)])
    val = jnp.concatenate([vals, jnp.zeros((pad,), jnp.float32)])
    return dst, src, val


def kernel(user_emb, item_emb, weight0, weight1, social_vals, inter_vals,
           social_idx, inter_idx):
    ue_lo, ue_hi = user_emb[:, :HALF], user_emb[:, HALF:]
    it_lo, it_hi = item_emb[:, :HALF], item_emb[:, HALF:]
    zeros = jnp.zeros((USERS, HALF), jnp.float32)

    sd, ss, sv = _pack_edges(social_idx, social_vals)
    td, ts, tv = _pack_edges(inter_idx, inter_vals)

    a_lo, a_hi = _spmm_halves(sd, ss, sv, ue_lo, ue_hi, zeros, zeros)
    u_lo, u_hi = _layer_matmul(a_lo, a_hi, ue_lo, ue_hi, weight0)
    a_lo, a_hi = _spmm_halves(sd, ss, sv, u_lo, u_hi, zeros, zeros)
    u_lo, u_hi = _layer_matmul(a_lo, a_hi, u_lo, u_hi, weight1)
    f_lo, f_hi = _spmm_halves(td, ts, tv, it_lo, it_hi, u_lo, u_hi)

    final_user = jnp.concatenate([f_lo, f_hi], axis=1)
    return (final_user, item_emb)
